# baseline (speedup 1.0000x reference)
import functools

import jax
import jax.numpy as jnp
from jax import lax
from jax.experimental import pallas as pl
from jax.experimental.pallas import tpu as pltpu
from jax.experimental.pallas import tpu_sc as plsc

N = 32768
D = 164
DP = 256
HANDS = 84

RB = 1024
NBLK = N // RB

MR, MC = 256, 128


def _clean_mask_body(x_ref, vals_ref, mask_ref):
    x = x_ref[...]
    cleaned = jnp.where(jnp.isnan(x), 0.0, x)
    col = lax.broadcasted_iota(jnp.int32, (RB, D), 1)
    s = jnp.sum(jnp.where(col < HANDS, jnp.abs(cleaned), 0.0), axis=1,
                keepdims=True)
    keep = (s != 0.0).astype(jnp.float32)
    vals_ref[:, :D] = cleaned * keep
    vals_ref[:, D:] = jnp.zeros((RB, DP - D), jnp.float32)
    mask_ref[...] = keep


def _dest_body(m_ref, dest_ref):
    m = m_ref[...]
    i1 = lax.broadcasted_iota(jnp.int32, (MC, MC), 0)
    j1 = lax.broadcasted_iota(jnp.int32, (MC, MC), 1)
    ut = (i1 <= j1).astype(jnp.float32)
    cs = jnp.dot(m, ut, preferred_element_type=jnp.float32)
    row_tot = cs[:, MC - 1:MC]
    i2 = lax.broadcasted_iota(jnp.int32, (MR, MR), 0)
    j2 = lax.broadcasted_iota(jnp.int32, (MR, MR), 1)
    slt = (j2 < i2).astype(jnp.float32)
    offs = jnp.dot(slt, row_tot, preferred_element_type=jnp.float32)
    csg = cs + offs
    total = offs[MR - 1:MR, 0:1] + row_tot[MR - 1:MR, 0:1]
    gi = (lax.broadcasted_iota(jnp.int32, (MR, MC), 0) * MC
          + lax.broadcasted_iota(jnp.int32, (MR, MC), 1)).astype(jnp.float32)
    cs_not = (gi + 1.0) - csg
    dest = jnp.where(m > 0.0, csg - 1.0, total + cs_not - 1.0)
    dest_ref[...] = dest.astype(jnp.int32)


_clean_mask = pl.pallas_call(
    _clean_mask_body,
    grid=(NBLK,),
    in_specs=[pl.BlockSpec((RB, D), lambda i: (i, 0))],
    out_specs=[
        pl.BlockSpec((RB, DP), lambda i: (i, 0)),
        pl.BlockSpec((RB, 1), lambda i: (i, 0)),
    ],
    out_shape=[
        jax.ShapeDtypeStruct((N, DP), jnp.float32),
        jax.ShapeDtypeStruct((N, 1), jnp.float32),
    ],
)

_dest = pl.pallas_call(
    _dest_body,
    out_shape=jax.ShapeDtypeStruct((MR, MC), jnp.int32),
)


UB = 4096


def _unpad_body(p_ref, o_ref):
    o_ref[...] = p_ref[:, :D]


_unpad = pl.pallas_call(
    _unpad_body,
    grid=(N // UB,),
    in_specs=[pl.BlockSpec((UB, DP), lambda i: (i, 0))],
    out_specs=pl.BlockSpec((UB, D), lambda i: (i, 0)),
    out_shape=jax.ShapeDtypeStruct((N, D), jnp.float32),
)


_NC, _NS = 2, 16
NW = _NC * _NS
RPW = N // NW
CH = 128
NCHUNK = RPW // CH


@functools.cache
def _make_sc_scatter():
    mesh = plsc.VectorSubcoreMesh(core_axis_name="c", subcore_axis_name="s")

    @functools.partial(
        pl.kernel,
        mesh=mesh,
        out_type=jax.ShapeDtypeStruct((N, DP), jnp.float32),
        scratch_types=[
            pltpu.VMEM((2, CH, DP), jnp.float32),
            pltpu.VMEM((2, CH), jnp.int32),
            pltpu.SemaphoreType.DMA,
            pltpu.SemaphoreType.DMA,
            pltpu.SemaphoreType.DMA,
        ],
    )
    def sc_scatter(vals_hbm, dest_hbm, out_hbm, rows_v, idx_v, sem_r, sem_i,
                   sem_s):
        wid = lax.axis_index("s") * _NC + lax.axis_index("c")
        base = wid * RPW

        def start_load(k, b):
            pltpu.async_copy(
                vals_hbm.at[pl.ds(base + k * CH, CH)],
                rows_v.at[b], sem_r)
            pltpu.async_copy(
                dest_hbm.at[pl.ds((base + k * CH) // 128, 1)],
                idx_v.at[pl.ds(b, 1)], sem_i)

        def wait_load(k, b):
            pltpu.make_async_copy(
                vals_hbm.at[pl.ds(base + k * CH, CH)],
                rows_v.at[b], sem_r).wait()
            pltpu.make_async_copy(
                dest_hbm.at[pl.ds((base + k * CH) // 128, 1)],
                idx_v.at[pl.ds(b, 1)], sem_i).wait()

        start_load(0, 0)
        for k in range(NCHUNK):
            b = k % 2
            wait_load(k, b)
            if k + 1 < NCHUNK:
                start_load(k + 1, 1 - b)
            pltpu.async_copy(rows_v.at[b], out_hbm.at[idx_v.at[b]],
                             sem_s).wait()

    return sc_scatter


def kernel(data0):
    vals, mask = _clean_mask(data0)
    dest = _dest(mask.reshape(MR, MC))
    out_pad = _make_sc_scatter()(vals, dest)
    return _unpad(out_pad)


# speedup vs baseline: 1.2702x; 1.2702x over previous
import functools

import jax
import jax.numpy as jnp
from jax import lax
from jax.experimental import pallas as pl
from jax.experimental.pallas import tpu as pltpu
from jax.experimental.pallas import tpu_sc as plsc

N = 32768
D = 164
DP = 256
HANDS = 84

RB = 1024
NBLK = N // RB

MR, MC = 256, 128


def _fused_body(x_ref, vals_ref, dest_ref, macc):
    i = pl.program_id(0)
    x = x_ref[...]
    cleaned = jnp.where(jnp.isnan(x), 0.0, x)
    col = lax.broadcasted_iota(jnp.int32, (RB, D), 1)
    s = jnp.sum(jnp.where(col < HANDS, jnp.abs(cleaned), 0.0), axis=1,
                keepdims=True)
    keep = (s != 0.0).astype(jnp.float32)
    vals_ref[:, :D] = cleaned * keep
    vals_ref[:, D:] = jnp.zeros((RB, DP - D), jnp.float32)
    macc[pl.ds(i * (RB // MC), RB // MC), :] = keep.reshape(RB // MC, MC)

    @pl.when(i == NBLK - 1)
    def _():
        m = macc[...]
        i1 = lax.broadcasted_iota(jnp.int32, (MC, MC), 0)
        j1 = lax.broadcasted_iota(jnp.int32, (MC, MC), 1)
        ut = (i1 <= j1).astype(jnp.float32)
        cs = jnp.dot(m, ut, preferred_element_type=jnp.float32)
        row_tot = cs[:, MC - 1:MC]
        i2 = lax.broadcasted_iota(jnp.int32, (MR, MR), 0)
        j2 = lax.broadcasted_iota(jnp.int32, (MR, MR), 1)
        slt = (j2 < i2).astype(jnp.float32)
        offs = jnp.dot(slt, row_tot,
                       preferred_element_type=jnp.float32)
        csg = cs + offs
        total = offs[MR - 1:MR, 0:1] + row_tot[MR - 1:MR, 0:1]
        gi = (lax.broadcasted_iota(jnp.int32, (MR, MC), 0) * MC
              + lax.broadcasted_iota(jnp.int32, (MR, MC), 1)
              ).astype(jnp.float32)
        cs_not = (gi + 1.0) - csg
        dest = jnp.where(m > 0.0, csg - 1.0, total + cs_not - 1.0)
        dest_ref[...] = dest.astype(jnp.int32)


_fused = pl.pallas_call(
    _fused_body,
    grid=(NBLK,),
    in_specs=[pl.BlockSpec((RB, D), lambda i: (i, 0))],
    out_specs=[
        pl.BlockSpec((RB, DP), lambda i: (i, 0)),
        pl.BlockSpec((MR, MC), lambda i: (0, 0)),
    ],
    out_shape=[
        jax.ShapeDtypeStruct((N, DP), jnp.float32),
        jax.ShapeDtypeStruct((MR, MC), jnp.int32),
    ],
    scratch_shapes=[pltpu.VMEM((MR, MC), jnp.float32)],
)


UB = 4096


def _unpad_body(p_ref, o_ref):
    o_ref[...] = p_ref[:, :D]


_unpad = pl.pallas_call(
    _unpad_body,
    grid=(N // UB,),
    in_specs=[pl.BlockSpec((UB, DP), lambda i: (i, 0))],
    out_specs=pl.BlockSpec((UB, D), lambda i: (i, 0)),
    out_shape=jax.ShapeDtypeStruct((N, D), jnp.float32),
)


_NC, _NS = 2, 16
NW = _NC * _NS
RPW = N // NW
CH = 128
NCHUNK = RPW // CH


@functools.cache
def _make_sc_scatter():
    mesh = plsc.VectorSubcoreMesh(core_axis_name="c", subcore_axis_name="s")

    @functools.partial(
        pl.kernel,
        mesh=mesh,
        out_type=jax.ShapeDtypeStruct((N, DP), jnp.float32),
        scratch_types=[
            pltpu.VMEM((2, CH, DP), jnp.float32),
            pltpu.VMEM((2, CH), jnp.int32),
            pltpu.SemaphoreType.DMA,
            pltpu.SemaphoreType.DMA,
            pltpu.SemaphoreType.DMA,
        ],
    )
    def sc_scatter(vals_hbm, dest_hbm, out_hbm, rows_v, idx_v, sem_r, sem_i,
                   sem_s):
        wid = lax.axis_index("s") * _NC + lax.axis_index("c")
        base = wid * RPW

        def start_load(k, b):
            pltpu.async_copy(
                vals_hbm.at[pl.ds(base + k * CH, CH)],
                rows_v.at[b], sem_r)
            pltpu.async_copy(
                dest_hbm.at[pl.ds((base + k * CH) // 128, 1)],
                idx_v.at[pl.ds(b, 1)], sem_i)

        def wait_load(k, b):
            pltpu.make_async_copy(
                vals_hbm.at[pl.ds(base + k * CH, CH)],
                rows_v.at[b], sem_r).wait()
            pltpu.make_async_copy(
                dest_hbm.at[pl.ds((base + k * CH) // 128, 1)],
                idx_v.at[pl.ds(b, 1)], sem_i).wait()

        start_load(0, 0)
        for k in range(NCHUNK):
            b = k % 2
            wait_load(k, b)
            if k + 1 < NCHUNK:
                start_load(k + 1, 1 - b)
            pltpu.async_copy(rows_v.at[b], out_hbm.at[idx_v.at[b]],
                             sem_s).wait()

    return sc_scatter


def kernel(data0):
    vals, dest = _fused(data0)
    out_pad = _make_sc_scatter()(vals, dest)
    return out_pad[:, :D]


# speedup vs baseline: 1.3737x; 1.0815x over previous
import functools

import jax
import jax.numpy as jnp
from jax import lax
from jax.experimental import pallas as pl
from jax.experimental.pallas import tpu as pltpu
from jax.experimental.pallas import tpu_sc as plsc

N = 32768
D = 164
DP = 256
HANDS = 84

RB = 2048
NBLK = N // RB

MR, MC = 256, 128


def _fused_body(x_ref, vals_ref, dest_ref, macc):
    i = pl.program_id(0)
    x = x_ref[...]
    cleaned = jnp.where(jnp.isnan(x), 0.0, x)
    col = lax.broadcasted_iota(jnp.int32, (RB, D), 1)
    s = jnp.sum(jnp.where(col < HANDS, jnp.abs(cleaned), 0.0), axis=1,
                keepdims=True)
    keep = (s != 0.0).astype(jnp.float32)
    vals_ref[:, :D] = cleaned * keep
    macc[pl.ds(i * (RB // MC), RB // MC), :] = keep.reshape(RB // MC, MC)

    @pl.when(i == NBLK - 1)
    def _():
        m = macc[...]
        i1 = lax.broadcasted_iota(jnp.int32, (MC, MC), 0)
        j1 = lax.broadcasted_iota(jnp.int32, (MC, MC), 1)
        ut = (i1 <= j1).astype(jnp.float32)
        cs = jnp.dot(m, ut, preferred_element_type=jnp.float32)
        row_tot = cs[:, MC - 1:MC]
        i2 = lax.broadcasted_iota(jnp.int32, (MR, MR), 0)
        j2 = lax.broadcasted_iota(jnp.int32, (MR, MR), 1)
        slt = (j2 < i2).astype(jnp.float32)
        offs = jnp.dot(slt, row_tot,
                       preferred_element_type=jnp.float32)
        csg = cs + offs
        total = offs[MR - 1:MR, 0:1] + row_tot[MR - 1:MR, 0:1]
        gi = (lax.broadcasted_iota(jnp.int32, (MR, MC), 0) * MC
              + lax.broadcasted_iota(jnp.int32, (MR, MC), 1)
              ).astype(jnp.float32)
        cs_not = (gi + 1.0) - csg
        dest = jnp.where(m > 0.0, csg - 1.0, total + cs_not - 1.0)
        dest_ref[...] = dest.astype(jnp.int32)


_fused = pl.pallas_call(
    _fused_body,
    grid=(NBLK,),
    in_specs=[pl.BlockSpec((RB, D), lambda i: (i, 0))],
    out_specs=[
        pl.BlockSpec((RB, DP), lambda i: (i, 0)),
        pl.BlockSpec((MR, MC), lambda i: (0, 0)),
    ],
    out_shape=[
        jax.ShapeDtypeStruct((N, DP), jnp.float32),
        jax.ShapeDtypeStruct((MR, MC), jnp.int32),
    ],
    scratch_shapes=[pltpu.VMEM((MR, MC), jnp.float32)],
)


UB = 4096


def _unpad_body(p_ref, o_ref):
    o_ref[...] = p_ref[:, :D]


_unpad = pl.pallas_call(
    _unpad_body,
    grid=(N // UB,),
    in_specs=[pl.BlockSpec((UB, DP), lambda i: (i, 0))],
    out_specs=pl.BlockSpec((UB, D), lambda i: (i, 0)),
    out_shape=jax.ShapeDtypeStruct((N, D), jnp.float32),
)


_NC, _NS = 2, 16
NW = _NC * _NS
RPW = N // NW
CH = 128
NCHUNK = RPW // CH


@functools.cache
def _make_sc_scatter():
    mesh = plsc.VectorSubcoreMesh(core_axis_name="c", subcore_axis_name="s")

    @functools.partial(
        pl.kernel,
        mesh=mesh,
        out_type=jax.ShapeDtypeStruct((N, DP), jnp.float32),
        scratch_types=[
            pltpu.VMEM((2, CH, DP), jnp.float32),
            pltpu.VMEM((2, CH), jnp.int32),
            pltpu.SemaphoreType.DMA,
            pltpu.SemaphoreType.DMA,
            pltpu.SemaphoreType.DMA,
        ],
    )
    def sc_scatter(vals_hbm, dest_hbm, out_hbm, rows_v, idx_v, sem_r, sem_i,
                   sem_s):
        wid = lax.axis_index("s") * _NC + lax.axis_index("c")
        base = wid * RPW

        def start_load(k, b):
            pltpu.async_copy(
                vals_hbm.at[pl.ds(base + k * CH, CH)],
                rows_v.at[b], sem_r)
            pltpu.async_copy(
                dest_hbm.at[pl.ds((base + k * CH) // 128, 1)],
                idx_v.at[pl.ds(b, 1)], sem_i)

        def wait_load(k, b):
            pltpu.make_async_copy(
                vals_hbm.at[pl.ds(base + k * CH, CH)],
                rows_v.at[b], sem_r).wait()
            pltpu.make_async_copy(
                dest_hbm.at[pl.ds((base + k * CH) // 128, 1)],
                idx_v.at[pl.ds(b, 1)], sem_i).wait()

        start_load(0, 0)
        for k in range(NCHUNK):
            b = k % 2
            wait_load(k, b)
            if k + 1 < NCHUNK:
                start_load(k + 1, 1 - b)
            pltpu.async_copy(rows_v.at[b], out_hbm.at[idx_v.at[b]],
                             sem_s).wait()

    return sc_scatter


def kernel(data0):
    vals, dest = _fused(data0)
    out_pad = _make_sc_scatter()(vals, dest)
    return out_pad[:, :D]


# speedup vs baseline: 1.4114x; 1.0275x over previous
import functools

import jax
import jax.numpy as jnp
from jax import lax
from jax.experimental import pallas as pl
from jax.experimental.pallas import tpu as pltpu
from jax.experimental.pallas import tpu_sc as plsc

N = 32768
D = 164
DP = 256
HANDS = 84

RB = 4096
NBLK = N // RB

MR, MC = 256, 128


def _fused_body(x_ref, vals_ref, dest_ref, macc):
    i = pl.program_id(0)
    x = x_ref[...]
    cleaned = jnp.where(jnp.isnan(x), 0.0, x)
    col = lax.broadcasted_iota(jnp.int32, (RB, D), 1)
    s = jnp.sum(jnp.where(col < HANDS, jnp.abs(cleaned), 0.0), axis=1,
                keepdims=True)
    keep = (s != 0.0).astype(jnp.float32)
    vals_ref[:, :D] = cleaned * keep
    macc[pl.ds(i * (RB // MC), RB // MC), :] = keep.reshape(RB // MC, MC)

    @pl.when(i == NBLK - 1)
    def _():
        m = macc[...]
        i1 = lax.broadcasted_iota(jnp.int32, (MC, MC), 0)
        j1 = lax.broadcasted_iota(jnp.int32, (MC, MC), 1)
        ut = (i1 <= j1).astype(jnp.float32)
        cs = jnp.dot(m, ut, preferred_element_type=jnp.float32)
        row_tot = cs[:, MC - 1:MC]
        i2 = lax.broadcasted_iota(jnp.int32, (MR, MR), 0)
        j2 = lax.broadcasted_iota(jnp.int32, (MR, MR), 1)
        slt = (j2 < i2).astype(jnp.float32)
        offs = jnp.dot(slt, row_tot,
                       preferred_element_type=jnp.float32)
        csg = cs + offs
        total = offs[MR - 1:MR, 0:1] + row_tot[MR - 1:MR, 0:1]
        gi = (lax.broadcasted_iota(jnp.int32, (MR, MC), 0) * MC
              + lax.broadcasted_iota(jnp.int32, (MR, MC), 1)
              ).astype(jnp.float32)
        cs_not = (gi + 1.0) - csg
        dest = jnp.where(m > 0.0, csg - 1.0, total + cs_not - 1.0)
        dest_ref[...] = dest.astype(jnp.int32)


_fused = pl.pallas_call(
    _fused_body,
    grid=(NBLK,),
    in_specs=[pl.BlockSpec((RB, D), lambda i: (i, 0))],
    out_specs=[
        pl.BlockSpec((RB, DP), lambda i: (i, 0)),
        pl.BlockSpec((MR, MC), lambda i: (0, 0)),
    ],
    out_shape=[
        jax.ShapeDtypeStruct((N, DP), jnp.float32),
        jax.ShapeDtypeStruct((MR, MC), jnp.int32),
    ],
    scratch_shapes=[pltpu.VMEM((MR, MC), jnp.float32)],
)


UB = 4096


def _unpad_body(p_ref, o_ref):
    o_ref[...] = p_ref[:, :D]


_unpad = pl.pallas_call(
    _unpad_body,
    grid=(N // UB,),
    in_specs=[pl.BlockSpec((UB, DP), lambda i: (i, 0))],
    out_specs=pl.BlockSpec((UB, D), lambda i: (i, 0)),
    out_shape=jax.ShapeDtypeStruct((N, D), jnp.float32),
)


_NC, _NS = 2, 16
NW = _NC * _NS
RPW = N // NW
CH = 128
NCHUNK = RPW // CH


@functools.cache
def _make_sc_scatter():
    mesh = plsc.VectorSubcoreMesh(core_axis_name="c", subcore_axis_name="s")

    @functools.partial(
        pl.kernel,
        mesh=mesh,
        out_type=jax.ShapeDtypeStruct((N, DP), jnp.float32),
        scratch_types=[
            pltpu.VMEM((2, CH, DP), jnp.float32),
            pltpu.VMEM((2, CH), jnp.int32),
            pltpu.SemaphoreType.DMA,
            pltpu.SemaphoreType.DMA,
            pltpu.SemaphoreType.DMA,
        ],
    )
    def sc_scatter(vals_hbm, dest_hbm, out_hbm, rows_v, idx_v, sem_r, sem_i,
                   sem_s):
        wid = lax.axis_index("s") * _NC + lax.axis_index("c")
        base = wid * RPW

        def start_load(k, b):
            pltpu.async_copy(
                vals_hbm.at[pl.ds(base + k * CH, CH)],
                rows_v.at[b], sem_r)
            pltpu.async_copy(
                dest_hbm.at[pl.ds((base + k * CH) // 128, 1)],
                idx_v.at[pl.ds(b, 1)], sem_i)

        def wait_load(k, b):
            pltpu.make_async_copy(
                vals_hbm.at[pl.ds(base + k * CH, CH)],
                rows_v.at[b], sem_r).wait()
            pltpu.make_async_copy(
                dest_hbm.at[pl.ds((base + k * CH) // 128, 1)],
                idx_v.at[pl.ds(b, 1)], sem_i).wait()

        start_load(0, 0)
        for k in range(NCHUNK):
            b = k % 2
            wait_load(k, b)
            if k + 1 < NCHUNK:
                start_load(k + 1, 1 - b)
            pltpu.async_copy(rows_v.at[b], out_hbm.at[idx_v.at[b]],
                             sem_s).wait()

    return sc_scatter


def kernel(data0):
    vals, dest = _fused(data0)
    out_pad = _make_sc_scatter()(vals, dest)
    return out_pad[:, :D]


# speedup vs baseline: 1.4171x; 1.0040x over previous
import functools

import jax
import jax.numpy as jnp
from jax import lax
from jax.experimental import pallas as pl
from jax.experimental.pallas import tpu as pltpu
from jax.experimental.pallas import tpu_sc as plsc

N = 32768
D = 164
DP = 256
HANDS = 84

RB = 8192
NBLK = N // RB

MR, MC = 256, 128


def _fused_body(x_ref, vals_ref, dest_ref, macc):
    i = pl.program_id(0)
    x = x_ref[...]
    cleaned = jnp.where(jnp.isnan(x), 0.0, x)
    col = lax.broadcasted_iota(jnp.int32, (RB, D), 1)
    s = jnp.sum(jnp.where(col < HANDS, jnp.abs(cleaned), 0.0), axis=1,
                keepdims=True)
    keep = (s != 0.0).astype(jnp.float32)
    vals_ref[:, :D] = cleaned * keep
    macc[pl.ds(i * (RB // MC), RB // MC), :] = keep.reshape(RB // MC, MC)

    @pl.when(i == NBLK - 1)
    def _():
        m = macc[...]
        i1 = lax.broadcasted_iota(jnp.int32, (MC, MC), 0)
        j1 = lax.broadcasted_iota(jnp.int32, (MC, MC), 1)
        ut = (i1 <= j1).astype(jnp.float32)
        cs = jnp.dot(m, ut, preferred_element_type=jnp.float32)
        row_tot = cs[:, MC - 1:MC]
        i2 = lax.broadcasted_iota(jnp.int32, (MR, MR), 0)
        j2 = lax.broadcasted_iota(jnp.int32, (MR, MR), 1)
        slt = (j2 < i2).astype(jnp.float32)
        offs = jnp.dot(slt, row_tot,
                       preferred_element_type=jnp.float32)
        csg = cs + offs
        total = offs[MR - 1:MR, 0:1] + row_tot[MR - 1:MR, 0:1]
        gi = (lax.broadcasted_iota(jnp.int32, (MR, MC), 0) * MC
              + lax.broadcasted_iota(jnp.int32, (MR, MC), 1)
              ).astype(jnp.float32)
        cs_not = (gi + 1.0) - csg
        dest = jnp.where(m > 0.0, csg - 1.0, total + cs_not - 1.0)
        dest_ref[...] = dest.astype(jnp.int32)


_fused = pl.pallas_call(
    _fused_body,
    grid=(NBLK,),
    in_specs=[pl.BlockSpec((RB, D), lambda i: (i, 0))],
    out_specs=[
        pl.BlockSpec((RB, DP), lambda i: (i, 0)),
        pl.BlockSpec((MR, MC), lambda i: (0, 0)),
    ],
    out_shape=[
        jax.ShapeDtypeStruct((N, DP), jnp.float32),
        jax.ShapeDtypeStruct((MR, MC), jnp.int32),
    ],
    scratch_shapes=[pltpu.VMEM((MR, MC), jnp.float32)],
)


UB = 4096


def _unpad_body(p_ref, o_ref):
    o_ref[...] = p_ref[:, :D]


_unpad = pl.pallas_call(
    _unpad_body,
    grid=(N // UB,),
    in_specs=[pl.BlockSpec((UB, DP), lambda i: (i, 0))],
    out_specs=pl.BlockSpec((UB, D), lambda i: (i, 0)),
    out_shape=jax.ShapeDtypeStruct((N, D), jnp.float32),
)


_NC, _NS = 2, 16
NW = _NC * _NS
RPW = N // NW
CH = 128
NCHUNK = RPW // CH


@functools.cache
def _make_sc_scatter():
    mesh = plsc.VectorSubcoreMesh(core_axis_name="c", subcore_axis_name="s")

    @functools.partial(
        pl.kernel,
        mesh=mesh,
        out_type=jax.ShapeDtypeStruct((N, DP), jnp.float32),
        scratch_types=[
            pltpu.VMEM((2, CH, DP), jnp.float32),
            pltpu.VMEM((2, CH), jnp.int32),
            pltpu.SemaphoreType.DMA,
            pltpu.SemaphoreType.DMA,
            pltpu.SemaphoreType.DMA,
        ],
    )
    def sc_scatter(vals_hbm, dest_hbm, out_hbm, rows_v, idx_v, sem_r, sem_i,
                   sem_s):
        wid = lax.axis_index("s") * _NC + lax.axis_index("c")
        base = wid * RPW

        def start_load(k, b):
            pltpu.async_copy(
                vals_hbm.at[pl.ds(base + k * CH, CH)],
                rows_v.at[b], sem_r)
            pltpu.async_copy(
                dest_hbm.at[pl.ds((base + k * CH) // 128, 1)],
                idx_v.at[pl.ds(b, 1)], sem_i)

        def wait_load(k, b):
            pltpu.make_async_copy(
                vals_hbm.at[pl.ds(base + k * CH, CH)],
                rows_v.at[b], sem_r).wait()
            pltpu.make_async_copy(
                dest_hbm.at[pl.ds((base + k * CH) // 128, 1)],
                idx_v.at[pl.ds(b, 1)], sem_i).wait()

        start_load(0, 0)
        for k in range(NCHUNK):
            b = k % 2
            wait_load(k, b)
            if k + 1 < NCHUNK:
                start_load(k + 1, 1 - b)
            pltpu.async_copy(rows_v.at[b], out_hbm.at[idx_v.at[b]],
                             sem_s).wait()

    return sc_scatter


def kernel(data0):
    vals, dest = _fused(data0)
    out_pad = _make_sc_scatter()(vals, dest)
    return out_pad[:, :D]


# speedup vs baseline: 1.4194x; 1.0016x over previous
import functools

import jax
import jax.numpy as jnp
from jax import lax
from jax.experimental import pallas as pl
from jax.experimental.pallas import tpu as pltpu
from jax.experimental.pallas import tpu_sc as plsc

N = 32768
D = 164
DP = 256
HANDS = 84

RB = 8192
NBLK = N // RB

MR, MC = 256, 128


def _fused_body(x_ref, vals_ref, dest_ref, macc):
    i = pl.program_id(0)
    x = x_ref[...]
    cleaned = jnp.where(jnp.isnan(x), 0.0, x)
    col = lax.broadcasted_iota(jnp.int32, (RB, D), 1)
    s = jnp.sum(jnp.where(col < HANDS, jnp.abs(cleaned), 0.0), axis=1,
                keepdims=True)
    keep = (s != 0.0).astype(jnp.float32)
    vals_ref[:, :D] = cleaned * keep
    macc[pl.ds(i * (RB // MC), RB // MC), :] = keep.reshape(RB // MC, MC)

    @pl.when(i == NBLK - 1)
    def _():
        m = macc[...]
        i1 = lax.broadcasted_iota(jnp.int32, (MC, MC), 0)
        j1 = lax.broadcasted_iota(jnp.int32, (MC, MC), 1)
        ut = (i1 <= j1).astype(jnp.float32)
        cs = jnp.dot(m, ut, preferred_element_type=jnp.float32)
        row_tot = cs[:, MC - 1:MC]
        i2 = lax.broadcasted_iota(jnp.int32, (MR, MR), 0)
        j2 = lax.broadcasted_iota(jnp.int32, (MR, MR), 1)
        slt = (j2 < i2).astype(jnp.float32)
        offs = jnp.dot(slt, row_tot,
                       preferred_element_type=jnp.float32)
        csg = cs + offs
        total = offs[MR - 1:MR, 0:1] + row_tot[MR - 1:MR, 0:1]
        gi = (lax.broadcasted_iota(jnp.int32, (MR, MC), 0) * MC
              + lax.broadcasted_iota(jnp.int32, (MR, MC), 1)
              ).astype(jnp.float32)
        cs_not = (gi + 1.0) - csg
        dest = jnp.where(m > 0.0, csg - 1.0, total + cs_not - 1.0)
        dest_ref[...] = dest.astype(jnp.int32)


_fused = pl.pallas_call(
    _fused_body,
    grid=(NBLK,),
    in_specs=[pl.BlockSpec((RB, D), lambda i: (i, 0))],
    out_specs=[
        pl.BlockSpec((RB, DP), lambda i: (i, 0)),
        pl.BlockSpec((MR, MC), lambda i: (0, 0)),
    ],
    out_shape=[
        jax.ShapeDtypeStruct((N, DP), jnp.float32),
        jax.ShapeDtypeStruct((MR, MC), jnp.int32),
    ],
    scratch_shapes=[pltpu.VMEM((MR, MC), jnp.float32)],
)


_NC, _NS = 2, 16
NW = _NC * _NS
RPW = N // NW
CH = 128
NCHUNK = RPW // CH


@functools.cache
def _make_sc_scatter():
    mesh = plsc.VectorSubcoreMesh(core_axis_name="c", subcore_axis_name="s")

    @functools.partial(
        pl.kernel,
        mesh=mesh,
        out_type=jax.ShapeDtypeStruct((N, DP), jnp.float32),
        scratch_types=[
            pltpu.VMEM((2, CH, DP), jnp.float32),
            pltpu.VMEM((2, CH), jnp.int32),
            pltpu.SemaphoreType.DMA,
            pltpu.SemaphoreType.DMA,
            pltpu.SemaphoreType.DMA,
        ],
    )
    def sc_scatter(vals_hbm, dest_hbm, out_hbm, rows_v, idx_v, sem_r, sem_i,
                   sem_s):
        wid = lax.axis_index("s") * _NC + lax.axis_index("c")
        base = wid * RPW

        def start_load(k, b):
            pltpu.async_copy(
                vals_hbm.at[pl.ds(base + k * CH, CH)],
                rows_v.at[b], sem_r)
            pltpu.async_copy(
                dest_hbm.at[pl.ds((base + k * CH) // 128, 1)],
                idx_v.at[pl.ds(b, 1)], sem_i)

        def wait_load(k, b):
            pltpu.make_async_copy(
                vals_hbm.at[pl.ds(base + k * CH, CH)],
                rows_v.at[b], sem_r).wait()
            pltpu.make_async_copy(
                dest_hbm.at[pl.ds((base + k * CH) // 128, 1)],
                idx_v.at[pl.ds(b, 1)], sem_i).wait()

        start_load(0, 0)
        for k in range(NCHUNK):
            b = k % 2
            wait_load(k, b)
            if k + 1 < NCHUNK:
                start_load(k + 1, 1 - b)
            pltpu.async_copy(rows_v.at[b], out_hbm.at[idx_v.at[b]],
                             sem_s).wait()

    return sc_scatter


def kernel(data0):
    vals, dest = _fused(data0)
    out_pad = _make_sc_scatter()(vals, dest)
    return out_pad[:, :D]
